# Initial kernel scaffold; baseline (speedup 1.0000x reference)
import jax
import jax.numpy as jnp
from jax.experimental import pallas as pl


def kernel(x, edge_index, edge_attr, batch, W1_0, b1_0, W2_0, b2_0, W1_1, b1_1, W2_1, b2_1, W1_2, b1_2, W2_2, b2_2, Wl, bl):
    raise NotImplementedError("write your pallas kernel here")


# speedup vs baseline: 1.7025x; 1.7025x over previous
import functools

import jax
import jax.numpy as jnp
from jax import lax
from jax.experimental import pallas as pl
from jax.experimental.pallas import tpu as pltpu
from jax.experimental.pallas import tpu_sc as plsc

_N = 50000
_E = 800000
_D = 64
_G = 64

_NC = 2
_NS = 16
_HALF = _N // _NC
_TRASH = _HALF
_ZSLICE = 1568
_SPM_ROWS = _NS * _ZSLICE
_LASTSLICE = _HALF - (_NS - 1) * _ZSLICE

_CHUNK = 128
_NCHUNK = _E // _CHUNK
_CPS = -(-_NCHUNK // _NS)

_PREC = lax.Precision.HIGHEST


def _dot(a, b):
    return lax.dot_general(a, b, (((1,), (0,)), ((), ())),
                           precision=_PREC, preferred_element_type=jnp.float32)


_BE = 2000


def _emlp_body(a_ref, w1_ref, b1_ref, w2_ref, b2_ref, o_ref):
    t = jnp.maximum(_dot(a_ref[...], w1_ref[...]) + b1_ref[...], 0.0)
    o_ref[...] = _dot(t, w2_ref[...]) + b2_ref[...]


def _edge_mlp(edge_attr, w1, b1, w2, b2):
    grid = _E // _BE
    return pl.pallas_call(
        _emlp_body,
        grid=(grid,),
        in_specs=[
            pl.BlockSpec((_BE, 16), lambda i: (i, 0)),
            pl.BlockSpec((16, _D), lambda i: (0, 0)),
            pl.BlockSpec((1, _D), lambda i: (0, 0)),
            pl.BlockSpec((_D, _D), lambda i: (0, 0)),
            pl.BlockSpec((1, _D), lambda i: (0, 0)),
        ],
        out_specs=pl.BlockSpec((_BE, _D), lambda i: (i, 0)),
        out_shape=jax.ShapeDtypeStruct((_E, _D), jnp.float32),
    )(edge_attr, w1, b1, w2, b2)


_BN = 2000


def _update_body(h_ref, a_ref, o_ref):
    o_ref[...] = jnp.maximum(h_ref[...] + a_ref[...], 0.0)


def _update(h, agg):
    grid = _N // _BN
    return pl.pallas_call(
        _update_body,
        grid=(grid,),
        in_specs=[
            pl.BlockSpec((_BN, _D), lambda i: (i, 0)),
            pl.BlockSpec((_BN, _D), lambda i: (i, 0)),
        ],
        out_specs=pl.BlockSpec((_BN, _D), lambda i: (i, 0)),
        out_shape=jax.ShapeDtypeStruct((_N, _D), jnp.float32),
    )(h, agg)


def _pool_body(h_ref, b_ref, wl_ref, bl_ref, o_ref, acc_ref):
    i = pl.program_id(0)

    @pl.when(i == 0)
    def _():
        acc_ref[...] = jnp.zeros_like(acc_ref)

    seg = b_ref[0]
    onehot = (lax.broadcasted_iota(jnp.int32, (_G, _BN), 0) == seg
              ).astype(jnp.float32)
    acc_ref[...] += _dot(onehot, h_ref[...])

    @pl.when(i == pl.num_programs(0) - 1)
    def _():
        o_ref[...] = _dot(acc_ref[...], wl_ref[...]) + bl_ref[...]


def _pool(h, batch3d, wl, bl):
    grid = _N // _BN
    return pl.pallas_call(
        _pool_body,
        grid=(grid,),
        in_specs=[
            pl.BlockSpec((_BN, _D), lambda i: (i, 0)),
            pl.BlockSpec((1, 1, _BN), lambda i: (i, 0, 0)),
            pl.BlockSpec((_D, 1), lambda i: (0, 0)),
            pl.BlockSpec((1, 1), lambda i: (0, 0)),
        ],
        out_specs=pl.BlockSpec((_G, 1), lambda i: (0, 0)),
        out_shape=jax.ShapeDtypeStruct((_G, 1), jnp.float32),
        scratch_shapes=[pltpu.VMEM((_G, _D), jnp.float32)],
    )(h, batch3d, wl, bl)


def _sc_body(h_hbm, src_hbm, dst_hbm, e_hbm, z_hbm, out_hbm,
             sidx, didx, rows, ev, accum, sem_g, sem_e):
    cid = lax.axis_index("c")
    sid = lax.axis_index("s")
    base_node = cid * _HALF

    zlo = sid * _ZSLICE
    pltpu.sync_copy(z_hbm.at[pl.ds(zlo, _ZSLICE)], accum.at[pl.ds(zlo, _ZSLICE)])
    plsc.subcore_barrier()

    @pl.loop(0, _CPS)
    def _(i):
        ck = sid * _CPS + i

        @pl.when(ck < _NCHUNK)
        def _():
            ebase = pl.multiple_of(ck * _CHUNK, _CHUNK)
            pltpu.sync_copy(src_hbm.at[pl.ds(ebase, _CHUNK)], sidx)
            pltpu.sync_copy(dst_hbm.at[pl.ds(ebase, _CHUNK)], didx.at[0])
            cg = pltpu.async_copy(h_hbm.at[sidx], rows, sem_g)
            ce = pltpu.async_copy(e_hbm.at[pl.ds(ebase, _CHUNK)], ev, sem_e)

            @pl.loop(0, _CHUNK // 16)
            def _(j):
                v = didx[0, pl.ds(j * 16, 16)]
                lo = v - base_node
                ok = (lo >= 0) & (lo < _HALF)
                didx[0, pl.ds(j * 16, 16)] = jnp.where(ok, lo, _TRASH)

            cg.wait()
            ce.wait()

            @pl.loop(0, _CHUNK)
            def _(r):
                for q in range(4):
                    sl = (r, pl.ds(q * 16, 16))
                    rows[sl] = jnp.maximum(rows[sl] + ev[sl], 0.0)

            pltpu.sync_copy(rows, accum.at[didx.at[0]], add=True)

    plsc.subcore_barrier()

    @pl.when(sid < _NS - 1)
    def _():
        pltpu.sync_copy(accum.at[pl.ds(sid * _ZSLICE, _ZSLICE)],
                        out_hbm.at[pl.ds(base_node + sid * _ZSLICE, _ZSLICE)])

    @pl.when(sid == _NS - 1)
    def _():
        pltpu.sync_copy(
            accum.at[pl.ds((_NS - 1) * _ZSLICE, _LASTSLICE)],
            out_hbm.at[pl.ds(base_node + (_NS - 1) * _ZSLICE, _LASTSLICE)])


@functools.cache
def _sc_pass_fn():
    mesh = plsc.VectorSubcoreMesh(core_axis_name="c", subcore_axis_name="s",
                                  num_cores=_NC, num_subcores=_NS)
    return pl.kernel(
        _sc_body,
        out_type=jax.ShapeDtypeStruct((_N, _D), jnp.float32),
        mesh=mesh,
        scratch_types=[
            pltpu.VMEM((_CHUNK,), jnp.int32),
            pltpu.VMEM((1, _CHUNK), jnp.int32),
            pltpu.VMEM((_CHUNK, _D), jnp.float32),
            pltpu.VMEM((_CHUNK, _D), jnp.float32),
            pltpu.VMEM_SHARED((_SPM_ROWS, _D), jnp.float32),
            pltpu.SemaphoreType.DMA,
            pltpu.SemaphoreType.DMA,
        ],
        compiler_params=pltpu.CompilerParams(use_tc_tiling_on_sc=False),
    )


def kernel(x, edge_index, edge_attr, batch,
           W1_0, b1_0, W2_0, b2_0,
           W1_1, b1_1, W2_1, b2_1,
           W1_2, b1_2, W2_2, b2_2,
           Wl, bl):
    src = edge_index[0]
    dst = edge_index[1]
    zeros = jnp.zeros((_SPM_ROWS, _D), jnp.float32)

    es = [
        _edge_mlp(edge_attr, W1_0, b1_0.reshape(1, -1), W2_0, b2_0.reshape(1, -1)),
        _edge_mlp(edge_attr, W1_1, b1_1.reshape(1, -1), W2_1, b2_1.reshape(1, -1)),
        _edge_mlp(edge_attr, W1_2, b1_2.reshape(1, -1), W2_2, b2_2.reshape(1, -1)),
    ]

    h = x
    for e in es:
        agg = _sc_pass_fn()(h, src, dst, e, zeros)
        h = _update(h, agg)

    out = _pool(h, batch.reshape(_N // _BN, 1, _BN), Wl, bl.reshape(1, 1))
    return jnp.squeeze(out, -1)
